# Initial kernel scaffold; baseline (speedup 1.0000x reference)
import jax
import jax.numpy as jnp
from jax.experimental import pallas as pl


def kernel(x, edge_index, eps, W0, b0, g_mlp, be_mlp, W1, b1, g_apply, be_apply, g_out, be_out):
    raise NotImplementedError("write your pallas kernel here")


# speedup vs baseline: 3.1899x; 3.1899x over previous
import functools

import jax
import jax.numpy as jnp
from jax import lax
from jax.experimental import pallas as pl
from jax.experimental.pallas import tpu as pltpu
from jax.experimental.pallas import tpu_sc as plsc

N = 10000
D = 256
E = 160000
NLAYER = 2

NC = 2
NS = 16
LANES = 16
HALF = D // NC

CH = 128
EPT = E // NS
NCHUNK = -(-EPT // CH)
EPT_PAD = NCHUNK * CH
E_PAD = EPT_PAD * NS
AGG_ROWS = ((N + CH - 1) // CH + 1) * CH
ZBLOCKS = AGG_ROWS // CH
ROWS_PER_TILE = AGG_ROWS // NS


def _agg_body(x2_hbm, idx_hbm, out_hbm, src_v, dst_v, rows_v, agg_s):
    c = lax.axis_index("c")
    t = lax.axis_index("s")

    pltpu.sync_copy(idx_hbm.at[c, t], src_v)
    pltpu.sync_copy(idx_hbm.at[2, t], dst_v)

    zero = jnp.zeros((LANES,), jnp.float32)

    def zrow(i, _):
        def zcol(j, _):
            rows_v[i, pl.ds(j * LANES, LANES)] = zero
            return 0
        return lax.fori_loop(0, HALF // LANES, zcol, 0)

    lax.fori_loop(0, CH, zrow, 0)

    def zblk(k, _):
        blk = t + k * NS

        @pl.when(blk < ZBLOCKS)
        def _():
            pltpu.sync_copy(rows_v, agg_s.at[pl.ds(blk * CH, CH)])
        return 0

    lax.fori_loop(0, -(-ZBLOCKS // NS), zblk, 0)
    plsc.subcore_barrier()

    def step(j, _):
        pltpu.sync_copy(x2_hbm.at[src_v.at[j]], rows_v)
        pltpu.sync_copy(rows_v, agg_s.at[dst_v.at[j]], add=True)
        return 0

    lax.fori_loop(0, NCHUNK, step, 0)
    plsc.subcore_barrier()

    r0 = t * ROWS_PER_TILE
    pltpu.sync_copy(agg_s.at[pl.ds(r0, ROWS_PER_TILE)],
                    out_hbm.at[c, pl.ds(r0, ROWS_PER_TILE)])


_agg_call = pl.kernel(
    _agg_body,
    out_type=jax.ShapeDtypeStruct((NC, AGG_ROWS, HALF), jnp.float32),
    mesh=plsc.VectorSubcoreMesh(core_axis_name="c", subcore_axis_name="s"),
    scratch_types=[
        pltpu.VMEM((NCHUNK, CH), jnp.int32),
        pltpu.VMEM((NCHUNK, CH), jnp.int32),
        pltpu.VMEM((CH, HALF), jnp.float32),
        pltpu.MemorySpace.VMEM_SHARED((AGG_ROWS, HALF), jnp.float32),
    ],
)


def _bn2(h, g, b):
    m = jnp.mean(h, axis=0, keepdims=True)
    v = jnp.mean((h - m) * (h - m), axis=0, keepdims=True)
    return g * (h - m) * lax.rsqrt(v + 1e-5) + b


def _dense_body(eps_ref, h_ref, agg_ref, W0_ref, b0_ref, gm_ref, bm_ref,
                W1_ref, b1_ref, ga_ref, ba_ref, go_ref, bo_ref, out_ref):
    h2 = (1.0 + eps_ref[0, 0]) * h_ref[...] + agg_ref[...]
    y = jnp.dot(h2, W0_ref[...], preferred_element_type=jnp.float32)
    y = jnp.maximum(_bn2(y + b0_ref[...], gm_ref[...], bm_ref[...]), 0.0)
    y = jnp.dot(y, W1_ref[...], preferred_element_type=jnp.float32)
    y = jnp.maximum(_bn2(y + b1_ref[...], ga_ref[...], ba_ref[...]), 0.0)
    out_ref[...] = _bn2(y, go_ref[...], bo_ref[...])


_dense_call = pl.pallas_call(
    _dense_body,
    out_shape=jax.ShapeDtypeStruct((N, D), jnp.float32),
)


def kernel(x, edge_index, eps, W0, b0, g_mlp, be_mlp, W1, b1,
           g_apply, be_apply, g_out, be_out):
    src = edge_index[0]
    dst = edge_index[1]
    idx3 = jnp.stack([2 * src, 2 * src + 1, dst])
    idx3 = jnp.pad(idx3, ((0, 0), (0, E_PAD - E)), constant_values=N)
    idx3 = idx3.reshape(3, NS, NCHUNK, CH)

    h = x
    for i in range(NLAYER):
        agg2 = _agg_call(h.reshape(2 * N, HALF), idx3)
        agg = agg2[:, :N].transpose(1, 0, 2).reshape(N, D)
        h = _dense_call(eps[i].reshape(1, 1), h, agg,
                        W0[i], b0[i].reshape(1, D),
                        g_mlp[i].reshape(1, D), be_mlp[i].reshape(1, D),
                        W1[i], b1[i].reshape(1, D),
                        g_apply[i].reshape(1, D), be_apply[i].reshape(1, D),
                        g_out[i].reshape(1, D), be_out[i].reshape(1, D))
    return h
